# baseline (speedup 1.0000x reference)
import jax
import jax.numpy as jnp
from jax import lax
from jax.experimental import pallas as pl
from jax.experimental.pallas import tpu as pltpu
from jax.experimental.pallas import tpu_sc as plsc

_B = 32
_C = 3
_H = 224
_W = 224
_RCHUNK = 56
_NCH = _H // _RCHUNK


def _sc_body(tsb_hbm, sa_hbm, so_hbm, x0_hbm, n_hbm, out_hbm,
             idx16, sa16, so16, xb, nb, ob, sem1, sem2):
    w = lax.axis_index("s") * 2 + lax.axis_index("c")
    pltpu.sync_copy(tsb_hbm.at[w], idx16)
    g1 = pltpu.async_copy(sa_hbm.at[idx16], sa16, sem1)
    g2 = pltpu.async_copy(so_hbm.at[idx16], so16, sem2)
    g1.wait()
    g2.wait()
    sa_v = sa16[...]
    so_v = so16[...]
    for ch in range(_C):
        for r in range(_NCH):
            cx = pltpu.async_copy(x0_hbm.at[w, ch, pl.ds(r * _RCHUNK, _RCHUNK)],
                                  xb, sem1)
            cn = pltpu.async_copy(n_hbm.at[w, ch, pl.ds(r * _RCHUNK, _RCHUNK)],
                                  nb, sem2)
            cx.wait()
            cn.wait()

            def _row(i, carry):
                for j in range(_W // 16):
                    sl = pl.ds(j * 16, 16)
                    ob[i, sl] = sa_v * xb[i, sl] + so_v * nb[i, sl]
                return carry

            lax.fori_loop(0, _RCHUNK, _row, 0)
            pltpu.sync_copy(ob, out_hbm.at[w, ch, pl.ds(r * _RCHUNK, _RCHUNK)])


@jax.jit
def kernel(x0, noise, time_steps, sqrt_alpha_cumprod, sqrt_one_minus_alpha_cumprod):
    ts_b = jnp.broadcast_to(time_steps.astype(jnp.int32)[:, None], (_B, 16))
    mesh = plsc.VectorSubcoreMesh(core_axis_name="c", subcore_axis_name="s")
    return pl.kernel(
        _sc_body,
        out_type=jax.ShapeDtypeStruct((_B, _C, _H, _W), jnp.float32),
        mesh=mesh,
        scratch_types=(
            pltpu.VMEM((16,), jnp.int32),
            pltpu.VMEM((16,), jnp.float32),
            pltpu.VMEM((16,), jnp.float32),
            pltpu.VMEM((_RCHUNK, _W), jnp.float32),
            pltpu.VMEM((_RCHUNK, _W), jnp.float32),
            pltpu.VMEM((_RCHUNK, _W), jnp.float32),
            pltpu.SemaphoreType.DMA,
            pltpu.SemaphoreType.DMA,
        ),
    )(ts_b, sqrt_alpha_cumprod, sqrt_one_minus_alpha_cumprod, x0, noise)
